# baseline (speedup 1.0000x reference)
import functools
import math

import jax
import jax.numpy as jnp
from jax import lax
from jax.experimental import pallas as pl
from jax.experimental.pallas import tpu as pltpu
from jax.experimental.pallas import tpu_sc as plsc

D = 64
SEQ = 50
LANES = 16
NVEC = D // LANES

NC = 2
NS = 16
NW = NC * NS

CHUNK = 800
GATHER = 100
NGATHER = CHUNK // GATHER


def _rsqrt16(x):
    xi = plsc.bitcast(x, jnp.int32)
    yi = jnp.int32(0x5F3759DF) - (xi >> 1)
    y = plsc.bitcast(yi, jnp.float32)
    half_x = x * 0.5
    for _ in range(3):
        y = y * (1.5 - half_x * y * y)
    return y


def _make_kernel(n_rows):
    rows_per_w = n_rows // NW
    n_chunks = rows_per_w // CHUNK
    mesh = plsc.VectorSubcoreMesh(core_axis_name="c", subcore_axis_name="s")

    @functools.partial(
        pl.kernel,
        mesh=mesh,
        compiler_params=pltpu.CompilerParams(
            needs_layout_passes=False, use_tc_tiling_on_sc=False
        ),
        out_type=jax.ShapeDtypeStruct((n_rows, D), jnp.float32),
        scratch_types=[
            pltpu.VMEM((NGATHER, GATHER), jnp.int32),
            pltpu.VMEM((CHUNK, D), jnp.float32),
            pltpu.VMEM((SEQ * D,), jnp.float32),
            pltpu.VMEM((D,), jnp.float32),
            pltpu.VMEM((D,), jnp.float32),
            pltpu.SemaphoreType.DMA,
        ],
    )
    def kern(idx_hbm, table_hbm, pe_hbm, scale_hbm, bias_hbm, out_hbm,
             idx_v, rows_v, pe_v, scale_v, bias_v, sem):
        wid = lax.axis_index("s") * NC + lax.axis_index("c")

        pltpu.sync_copy(pe_hbm, pe_v)
        pltpu.sync_copy(scale_hbm, scale_v)
        pltpu.sync_copy(bias_hbm, bias_v)

        scale = [scale_v[pl.ds(k * LANES, LANES)] for k in range(NVEC)]
        bias = [bias_v[pl.ds(k * LANES, LANES)] for k in range(NVEC)]

        def do_chunk(ci, _):
            base = wid * rows_per_w + ci * CHUNK
            idx_row = pl.multiple_of(base // GATHER, 8)
            pltpu.sync_copy(idx_hbm.at[pl.ds(idx_row, NGATHER)], idx_v)
            cps = [
                pltpu.async_copy(
                    table_hbm.at[idx_v.at[g]],
                    rows_v.at[pl.ds(g * GATHER, GATHER)],
                    sem,
                )
                for g in range(NGATHER)
            ]
            for cp in cps:
                cp.wait()

            def do_row(j, _):
                pos = lax.rem(j, SEQ)
                row = rows_v.at[j]
                pebase = pos * D
                e = [
                    row[pl.ds(k * LANES, LANES)]
                    + pe_v[pl.ds(pebase + k * LANES, LANES)]
                    for k in range(NVEC)
                ]
                s = e[0] + e[1] + e[2] + e[3]
                q = e[0] * e[0] + e[1] * e[1] + e[2] * e[2] + e[3] * e[3]
                tot = jnp.sum(s)
                qtot = jnp.sum(q)
                mean = tot * (1.0 / D)
                var = qtot * (1.0 / D) - mean * mean
                inv = _rsqrt16(jnp.full((LANES,), var + 1e-5, jnp.float32))
                mean_v = jnp.full((LANES,), mean, jnp.float32)
                for k in range(NVEC):
                    row[pl.ds(k * LANES, LANES)] = (
                        (e[k] - mean_v) * inv * scale[k] + bias[k]
                    )
                return ()

            lax.fori_loop(0, CHUNK, do_row, (), unroll=False)
            pltpu.sync_copy(rows_v, out_hbm.at[pl.ds(base, CHUNK)])
            return ()

        lax.fori_loop(0, n_chunks, do_chunk, (), unroll=False)

    return kern


@jax.jit
def kernel(x, tok_embed, pe, norm_scale, norm_bias):
    b, s = x.shape
    n_rows = b * s
    idx = x.reshape(n_rows // GATHER, GATHER).astype(jnp.int32)
    pe_flat = pe.reshape(-1)[: SEQ * D].astype(jnp.float32)
    out = _make_kernel(n_rows)(
        idx, tok_embed, pe_flat,
        norm_scale.astype(jnp.float32), norm_bias.astype(jnp.float32),
    )
    return out.reshape(b, s, D)


# speedup vs baseline: 1.3186x; 1.3186x over previous
import functools

import jax
import jax.numpy as jnp
from jax import lax
from jax.experimental import pallas as pl
from jax.experimental.pallas import tpu as pltpu
from jax.experimental.pallas import tpu_sc as plsc

D = 64
SEQ = 50
LANES = 16
NVEC = D // LANES

NC = 2
NS = 16
NW = NC * NS

CB = 16
CHUNK = CB * SEQ


def _rsqrt16(x):
    xi = plsc.bitcast(x, jnp.int32)
    yi = jnp.int32(0x5F3759DF) - (xi >> 1)
    y = plsc.bitcast(yi, jnp.float32)
    half_x = x * 0.5
    for _ in range(3):
        y = y * (1.5 - half_x * y * y)
    return y


def _make_kernel(n_batch):
    b_per_w = n_batch // NW
    n_chunks = b_per_w // CB
    mesh = plsc.VectorSubcoreMesh(core_axis_name="c", subcore_axis_name="s")

    @functools.partial(
        pl.kernel,
        mesh=mesh,
        compiler_params=pltpu.CompilerParams(
            needs_layout_passes=False, use_tc_tiling_on_sc=False
        ),
        out_type=jax.ShapeDtypeStruct((n_batch, SEQ, D), jnp.float32),
        scratch_types=[
            pltpu.VMEM((2, CB, SEQ), jnp.int32),
            pltpu.VMEM((CB, SEQ, D), jnp.float32),
            pltpu.VMEM((CB, SEQ, D), jnp.float32),
            pltpu.VMEM((SEQ * D,), jnp.float32),
            pltpu.VMEM((D,), jnp.float32),
            pltpu.VMEM((D,), jnp.float32),
            pltpu.SemaphoreType.DMA,
            pltpu.SemaphoreType.DMA,
            pltpu.SemaphoreType.DMA,
            pltpu.SemaphoreType.DMA,
        ],
    )
    def kern(idx_hbm, table_hbm, pe_hbm, scale_hbm, bias_hbm, out_hbm,
             idx_v, rows0, rows1, pe_v, scale_v, bias_v,
             gsem0, gsem1, wsem0, wsem1):
        wid = lax.axis_index("s") * NC + lax.axis_index("c")
        rows = (rows0, rows1)
        gsem = (gsem0, gsem1)
        wsem = (wsem0, wsem1)

        pltpu.sync_copy(pe_hbm, pe_v)
        pltpu.sync_copy(scale_hbm, scale_v)
        pltpu.sync_copy(bias_hbm, bias_v)

        scale = [scale_v[pl.ds(k * LANES, LANES)] for k in range(NVEC)]
        bias = [bias_v[pl.ds(k * LANES, LANES)] for k in range(NVEC)]

        def stage(ci, buf):
            bb = pl.multiple_of(wid * b_per_w + ci * CB, 8)
            pltpu.sync_copy(idx_hbm.at[pl.ds(bb, CB)], idx_v.at[buf])
            cps = [
                pltpu.async_copy(
                    table_hbm.at[idx_v.at[buf, b]],
                    rows[buf].at[b],
                    gsem[buf],
                )
                for b in range(CB)
            ]
            return cps, bb

        def compute(buf):
            rbuf = rows[buf]

            @plsc.parallel_loop(0, CHUNK, 1, unroll=4)
            def _(j):
                jb = lax.div(j, SEQ)
                js = lax.rem(j, SEQ)
                row = rbuf.at[jb, js]
                pebase = js * D
                e = [
                    row[pl.ds(k * LANES, LANES)]
                    + pe_v[pl.ds(pebase + k * LANES, LANES)]
                    for k in range(NVEC)
                ]
                s = e[0] + e[1] + e[2] + e[3]
                q = e[0] * e[0] + e[1] * e[1] + e[2] * e[2] + e[3] * e[3]
                tot = jnp.sum(s)
                qtot = jnp.sum(q)
                mean = tot * (1.0 / D)
                var = qtot * (1.0 / D) - mean * mean
                inv = _rsqrt16(jnp.full((LANES,), var + 1e-5, jnp.float32))
                mean_v = jnp.full((LANES,), mean, jnp.float32)
                for k in range(NVEC):
                    row[pl.ds(k * LANES, LANES)] = (
                        (e[k] - mean_v) * inv * scale[k] + bias[k]
                    )

        pend = {0: stage(0, 0)}
        wcp = [None, None]
        for ci in range(n_chunks):
            cur = ci & 1
            nxt = 1 - cur
            if ci + 1 < n_chunks:
                if wcp[nxt] is not None:
                    wcp[nxt].wait()
                    wcp[nxt] = None
                pend[nxt] = stage(ci + 1, nxt)
            cps, bb = pend[cur]
            for cp in cps:
                cp.wait()
            compute(cur)
            wcp[cur] = pltpu.async_copy(
                rows[cur], out_hbm.at[pl.ds(bb, CB)], wsem[cur]
            )
        for w in wcp:
            if w is not None:
                w.wait()

    return kern


@jax.jit
def kernel(x, tok_embed, pe, norm_scale, norm_bias):
    b, s = x.shape
    idx = x.astype(jnp.int32)
    pe_flat = pe.reshape(-1)[: SEQ * D].astype(jnp.float32)
    return _make_kernel(b)(
        idx, tok_embed, pe_flat,
        norm_scale.astype(jnp.float32), norm_bias.astype(jnp.float32),
    )
